# baseline (speedup 1.0000x reference)
import functools

import jax
import jax.numpy as jnp
from jax import lax
from jax.experimental import pallas as pl
from jax.experimental.pallas import tpu as pltpu
from jax.experimental.pallas import tpu_sc as plsc

B = 4096
N_USERS = 10000
N_ITEMS = 5000
D = 16
TW = 128

SC_CORES = 2
SC_SUBCORES = 16
NW = SC_CORES * SC_SUBCORES
BPW = B // NW

ROW_BLK = 256
N_ROW_BLKS = (N_ITEMS + 1 + ROW_BLK - 1) // ROW_BLK
ROWS_PAD = N_ROW_BLKS * ROW_BLK
W_COLS = N_USERS + 1


def _sweep_body(a_ref, wiT_ref, wuT_ref, iproj_ref, uproj_ref):
    half = pl.program_id(0)
    step = pl.program_id(1)
    a = a_ref[...]
    blk = half * (N_ROW_BLKS // 2) + step
    row_ids = blk * ROW_BLK + lax.broadcasted_iota(jnp.int32, (ROW_BLK, 1), 0)
    a = jnp.where(row_ids < N_ITEMS, a, 0.0)
    iproj_ref[...] = jnp.dot(a, wiT_ref[...], preferred_element_type=jnp.float32)
    contrib = lax.dot_general(
        a, wuT_ref[...], (((0,), (0,)), ((), ())),
        preferred_element_type=jnp.float32)[None]

    @pl.when(step == 0)
    def _():
        uproj_ref[...] = contrib

    @pl.when(step != 0)
    def _():
        uproj_ref[...] += contrib


def _projections(at, wiT_pad, wuT_pad):
    return pl.pallas_call(
        _sweep_body,
        grid=(2, N_ROW_BLKS // 2),
        in_specs=[
            pl.BlockSpec((ROW_BLK, W_COLS),
                         lambda j, i: (j * (N_ROW_BLKS // 2) + i, 0)),
            pl.BlockSpec((W_COLS, D), lambda j, i: (0, 0)),
            pl.BlockSpec((ROW_BLK, D),
                         lambda j, i: (j * (N_ROW_BLKS // 2) + i, 0)),
        ],
        out_specs=[
            pl.BlockSpec((ROW_BLK, D),
                         lambda j, i: (j * (N_ROW_BLKS // 2) + i, 0)),
            pl.BlockSpec((1, W_COLS, D), lambda j, i: (j, 0, 0)),
        ],
        out_shape=[
            jax.ShapeDtypeStruct((ROWS_PAD, D), jnp.float32),
            jax.ShapeDtypeStruct((2, W_COLS, D), jnp.float32),
        ],
        compiler_params=pltpu.CompilerParams(
            dimension_semantics=("parallel", "arbitrary")),
    )(at, wiT_pad, wuT_pad)


def _sc_delta_body(uidx_hbm, iidx_hbm, at_hbm, out_w,
                   uidx_v, iidx_v, w_v, wsem):
    wid = lax.axis_index("s") * SC_CORES + lax.axis_index("c")
    base = wid * BPW
    pltpu.sync_copy(uidx_hbm.at[pl.ds(base, BPW)], uidx_v)
    pltpu.sync_copy(iidx_hbm.at[pl.ds(base, BPW)], iidx_v)

    def chunk(j):
        off = pl.multiple_of(j * 16, 16)
        u16 = uidx_v[pl.ds(off, 16)]
        i16 = iidx_v[pl.ds(off, 16)]
        r0 = (i16 >> 3) << 3
        c0 = (u16 >> 7) << 7
        sub = i16 & 7
        waits = []
        for k in range(16):
            r_s = pl.multiple_of(r0[k], 8)
            c_s = pl.multiple_of(c0[k], 128)
            waits.append(pltpu.async_copy(
                at_hbm.at[pl.ds(r_s, 8), pl.ds(c_s, 128)],
                w_v.at[pl.ds(k * 8, 8)], wsem))
        for c in waits:
            c.wait()
        for k in range(16):
            pltpu.sync_copy(w_v.at[pl.ds(k * 8 + sub[k], 1)],
                            out_w.at[pl.ds(base + off + k, 1)])

    pl.loop(0, BPW // 16)(chunk)


@functools.cache
def _sc_delta_kernel():
    return functools.partial(
        pl.kernel,
        mesh=plsc.VectorSubcoreMesh(core_axis_name="c", subcore_axis_name="s"),
        out_type=[
            jax.ShapeDtypeStruct((B, TW), jnp.float32),
        ],
        scratch_types=[
            pltpu.VMEM((BPW,), jnp.int32),
            pltpu.VMEM((BPW,), jnp.int32),
            pltpu.VMEM((128, TW), jnp.float32),
            pltpu.SemaphoreType.DMA,
        ],
    )(_sc_delta_body)


def _sc_rows_body(uidx_hbm, iidx_hbm, utab_hbm, itab_hbm,
                  out_u, out_i,
                  uidx_v, iidx_v, r_u, r_i, sem):
    wid = lax.axis_index("s") * SC_CORES + lax.axis_index("c")
    base = wid * BPW
    pltpu.sync_copy(uidx_hbm.at[pl.ds(base, BPW)], uidx_v)
    pltpu.sync_copy(iidx_hbm.at[pl.ds(base, BPW)], iidx_v)

    cu = pltpu.async_copy(utab_hbm.at[uidx_v], r_u, sem)
    ci = pltpu.async_copy(itab_hbm.at[iidx_v], r_i, sem)
    cu.wait()
    ci.wait()
    pltpu.sync_copy(r_u, out_u.at[pl.ds(base, BPW)])
    pltpu.sync_copy(r_i, out_i.at[pl.ds(base, BPW)])


@functools.cache
def _sc_rows_kernel():
    return functools.partial(
        pl.kernel,
        mesh=plsc.VectorSubcoreMesh(core_axis_name="c", subcore_axis_name="s"),
        out_type=[
            jax.ShapeDtypeStruct((B, TW), jnp.float32),
            jax.ShapeDtypeStruct((B, TW), jnp.float32),
        ],
        scratch_types=[
            pltpu.VMEM((BPW,), jnp.int32),
            pltpu.VMEM((BPW,), jnp.int32),
            pltpu.VMEM((BPW, TW), jnp.float32),
            pltpu.VMEM((BPW, TW), jnp.float32),
            pltpu.SemaphoreType.DMA,
        ],
    )(_sc_rows_body)


def _mlp_body(gu_ref, gi_ref, w3_ref, ui_ref, ii_ref,
              w1_ref, b1_ref, g1_ref, be1_ref, wl_ref, out_ref):
    ui = ui_ref[...]
    ln = lax.broadcasted_iota(jnp.int32, (B, TW), 1)
    oh = ln == (ui & 127)
    delta = jnp.sum(jnp.where(oh, w3_ref[...], 0.0), axis=-1, keepdims=True)
    gu = gu_ref[...]
    gi = gi_ref[...]
    proj_user = gu[:, D:2 * D] - delta * gi[:, 2 * D:3 * D]
    proj_item = gi[:, D:2 * D] - delta * gu[:, 2 * D:3 * D]
    x = jnp.concatenate([gu[:, :D], proj_user, gi[:, :D], proj_item], axis=-1)
    h = lax.dot_general(x, w1_ref[...], (((1,), (1,)), ((), ())),
                        preferred_element_type=jnp.float32) + b1_ref[...]
    mu = jnp.mean(h, axis=-1, keepdims=True)
    var = jnp.mean(jnp.square(h - mu), axis=-1, keepdims=True)
    h = (h - mu) * lax.rsqrt(var + 1e-5) * g1_ref[...] + be1_ref[...]
    h = jnp.maximum(h, 0.0)
    out_ref[...] = lax.dot_general(h, wl_ref[...], (((1,), (1,)), ((), ())),
                                   preferred_element_type=jnp.float32)


def _mlp(gu, gi, w3, ui, ii, W1, b1, g1, be1, Wl):
    return pl.pallas_call(
        _mlp_body,
        out_shape=jax.ShapeDtypeStruct((B, 1), jnp.float32),
    )(gu, gi, w3, ui, ii, W1, b1, g1, be1, Wl)


def kernel(user_idx, item_idx, interactions, user_emb, item_emb, Wu, Wi,
           W1, b1, g1, be1, Wl, bl):
    at = interactions.T

    wiT_pad = jnp.zeros((W_COLS, D), jnp.float32).at[:N_USERS].set(Wi.T)
    wuT_pad = jnp.zeros((ROWS_PAD, D), jnp.float32).at[:N_ITEMS].set(Wu.T)

    iproj, uproj_halves = _projections(at, wiT_pad, wuT_pad)
    uproj = uproj_halves[0] + uproj_halves[1]

    zu = jnp.zeros((N_USERS, TW - 3 * D), jnp.float32)
    utab = jnp.concatenate(
        [user_emb[:N_USERS], uproj[:N_USERS], wiT_pad[:N_USERS], zu], axis=1)
    zi = jnp.zeros((N_ITEMS, TW - 3 * D), jnp.float32)
    itab = jnp.concatenate(
        [item_emb[:N_ITEMS], iproj[:N_ITEMS], wuT_pad[:N_ITEMS], zi], axis=1)

    uidx = user_idx.astype(jnp.int32)
    iidx = item_idx.astype(jnp.int32)
    (w_rows,) = _sc_delta_kernel()(uidx, iidx, at)
    gu, gi = _sc_rows_kernel()(uidx, iidx, utab, itab)

    logit = _mlp(gu, gi, w_rows,
                 uidx.reshape(B, 1), iidx.reshape(B, 1),
                 W1, b1.reshape(1, 32), g1.reshape(1, 32),
                 be1.reshape(1, 32), Wl)
    return logit.reshape(B) + bl


# speedup vs baseline: 1.0789x; 1.0789x over previous
import functools

import jax
import jax.numpy as jnp
from jax import lax
from jax.experimental import pallas as pl
from jax.experimental.pallas import tpu as pltpu
from jax.experimental.pallas import tpu_sc as plsc

B = 4096
N_USERS = 10000
N_ITEMS = 5000
D = 16
TW = 128

SC_CORES = 2
SC_SUBCORES = 16
NW = SC_CORES * SC_SUBCORES
BPW = B // NW

ROW_BLK = 512
N_ROW_BLKS = (N_ITEMS + 1 + ROW_BLK - 1) // ROW_BLK
ROWS_PAD = N_ROW_BLKS * ROW_BLK
W_COLS = N_USERS + 1


def _sweep_body(a_ref, wiT_ref, wuT_ref, iproj_ref, uproj_ref):
    step = pl.program_id(0)
    a = a_ref[...]
    row_ids = step * ROW_BLK + lax.broadcasted_iota(jnp.int32, (ROW_BLK, 1), 0)
    a = jnp.where(row_ids < N_ITEMS, a, 0.0)
    iproj_ref[...] = jnp.dot(a, wiT_ref[...], preferred_element_type=jnp.float32)
    contrib = lax.dot_general(
        a, wuT_ref[...], (((0,), (0,)), ((), ())),
        preferred_element_type=jnp.float32)

    @pl.when(step == 0)
    def _():
        uproj_ref[...] = contrib

    @pl.when(step != 0)
    def _():
        uproj_ref[...] += contrib


def _projections(at, wiT_pad, wuT_pad):
    return pl.pallas_call(
        _sweep_body,
        grid=(N_ROW_BLKS,),
        in_specs=[
            pl.BlockSpec((ROW_BLK, W_COLS), lambda i: (i, 0)),
            pl.BlockSpec((W_COLS, D), lambda i: (0, 0)),
            pl.BlockSpec((ROW_BLK, D), lambda i: (i, 0)),
        ],
        out_specs=[
            pl.BlockSpec((ROW_BLK, D), lambda i: (i, 0)),
            pl.BlockSpec((W_COLS, D), lambda i: (0, 0)),
        ],
        out_shape=[
            jax.ShapeDtypeStruct((ROWS_PAD, D), jnp.float32),
            jax.ShapeDtypeStruct((W_COLS, D), jnp.float32),
        ],
        compiler_params=pltpu.CompilerParams(
            dimension_semantics=("arbitrary",)),
    )(at, wiT_pad, wuT_pad)


def _sc_delta_body(uidx_hbm, iidx_hbm, at_hbm, out_w,
                   uidx_v, iidx_v, w_v, wsem):
    wid = lax.axis_index("s") * SC_CORES + lax.axis_index("c")
    base = wid * BPW
    pltpu.sync_copy(uidx_hbm.at[pl.ds(base, BPW)], uidx_v)
    pltpu.sync_copy(iidx_hbm.at[pl.ds(base, BPW)], iidx_v)

    def chunk(j):
        off = pl.multiple_of(j * 16, 16)
        u16 = uidx_v[pl.ds(off, 16)]
        i16 = iidx_v[pl.ds(off, 16)]
        r0 = (i16 >> 3) << 3
        c0 = (u16 >> 7) << 7
        sub = i16 & 7
        waits = []
        for k in range(16):
            r_s = pl.multiple_of(r0[k], 8)
            c_s = pl.multiple_of(c0[k], 128)
            waits.append(pltpu.async_copy(
                at_hbm.at[pl.ds(r_s, 8), pl.ds(c_s, 128)],
                w_v.at[pl.ds(k * 8, 8)], wsem))
        for c in waits:
            c.wait()
        for k in range(16):
            pltpu.sync_copy(w_v.at[pl.ds(k * 8 + sub[k], 1)],
                            out_w.at[pl.ds(base + off + k, 1)])

    pl.loop(0, BPW // 16)(chunk)


@functools.cache
def _sc_delta_kernel():
    return functools.partial(
        pl.kernel,
        mesh=plsc.VectorSubcoreMesh(core_axis_name="c", subcore_axis_name="s"),
        out_type=[
            jax.ShapeDtypeStruct((B, TW), jnp.float32),
        ],
        scratch_types=[
            pltpu.VMEM((BPW,), jnp.int32),
            pltpu.VMEM((BPW,), jnp.int32),
            pltpu.VMEM((128, TW), jnp.float32),
            pltpu.SemaphoreType.DMA,
        ],
    )(_sc_delta_body)


def _sc_rows_body(uidx_hbm, iidx_hbm, utab_hbm, itab_hbm,
                  out_u, out_i,
                  uidx_v, iidx_v, r_u, r_i, sem):
    wid = lax.axis_index("s") * SC_CORES + lax.axis_index("c")
    base = wid * BPW
    pltpu.sync_copy(uidx_hbm.at[pl.ds(base, BPW)], uidx_v)
    pltpu.sync_copy(iidx_hbm.at[pl.ds(base, BPW)], iidx_v)

    cu = pltpu.async_copy(utab_hbm.at[uidx_v], r_u, sem)
    ci = pltpu.async_copy(itab_hbm.at[iidx_v], r_i, sem)
    cu.wait()
    ci.wait()
    pltpu.sync_copy(r_u, out_u.at[pl.ds(base, BPW)])
    pltpu.sync_copy(r_i, out_i.at[pl.ds(base, BPW)])


@functools.cache
def _sc_rows_kernel():
    return functools.partial(
        pl.kernel,
        mesh=plsc.VectorSubcoreMesh(core_axis_name="c", subcore_axis_name="s"),
        out_type=[
            jax.ShapeDtypeStruct((B, TW), jnp.float32),
            jax.ShapeDtypeStruct((B, TW), jnp.float32),
        ],
        scratch_types=[
            pltpu.VMEM((BPW,), jnp.int32),
            pltpu.VMEM((BPW,), jnp.int32),
            pltpu.VMEM((BPW, TW), jnp.float32),
            pltpu.VMEM((BPW, TW), jnp.float32),
            pltpu.SemaphoreType.DMA,
        ],
    )(_sc_rows_body)


def _mlp_body(gu_ref, gi_ref, w3_ref, ui_ref, ii_ref,
              w1_ref, b1_ref, g1_ref, be1_ref, wl_ref, out_ref):
    ui = ui_ref[...]
    ln = lax.broadcasted_iota(jnp.int32, (B, TW), 1)
    oh = ln == (ui & 127)
    delta = jnp.sum(jnp.where(oh, w3_ref[...], 0.0), axis=-1, keepdims=True)
    gu = gu_ref[...]
    gi = gi_ref[...]
    proj_user = gu[:, D:2 * D] - delta * gi[:, 2 * D:3 * D]
    proj_item = gi[:, D:2 * D] - delta * gu[:, 2 * D:3 * D]
    x = jnp.concatenate([gu[:, :D], proj_user, gi[:, :D], proj_item], axis=-1)
    h = lax.dot_general(x, w1_ref[...], (((1,), (1,)), ((), ())),
                        preferred_element_type=jnp.float32) + b1_ref[...]
    mu = jnp.mean(h, axis=-1, keepdims=True)
    var = jnp.mean(jnp.square(h - mu), axis=-1, keepdims=True)
    h = (h - mu) * lax.rsqrt(var + 1e-5) * g1_ref[...] + be1_ref[...]
    h = jnp.maximum(h, 0.0)
    out_ref[...] = lax.dot_general(h, wl_ref[...], (((1,), (1,)), ((), ())),
                                   preferred_element_type=jnp.float32)


def _mlp(gu, gi, w3, ui, ii, W1, b1, g1, be1, Wl):
    return pl.pallas_call(
        _mlp_body,
        out_shape=jax.ShapeDtypeStruct((B, 1), jnp.float32),
    )(gu, gi, w3, ui, ii, W1, b1, g1, be1, Wl)


def kernel(user_idx, item_idx, interactions, user_emb, item_emb, Wu, Wi,
           W1, b1, g1, be1, Wl, bl):
    at = interactions.T

    wiT_pad = jnp.zeros((W_COLS, D), jnp.float32).at[:N_USERS].set(Wi.T)
    wuT_pad = jnp.zeros((ROWS_PAD, D), jnp.float32).at[:N_ITEMS].set(Wu.T)

    iproj, uproj = _projections(at, wiT_pad, wuT_pad)

    zu = jnp.zeros((N_USERS, TW - 3 * D), jnp.float32)
    utab = jnp.concatenate(
        [user_emb[:N_USERS], uproj[:N_USERS], wiT_pad[:N_USERS], zu], axis=1)
    zi = jnp.zeros((N_ITEMS, TW - 3 * D), jnp.float32)
    itab = jnp.concatenate(
        [item_emb[:N_ITEMS], iproj[:N_ITEMS], wuT_pad[:N_ITEMS], zi], axis=1)

    uidx = user_idx.astype(jnp.int32)
    iidx = item_idx.astype(jnp.int32)
    (w_rows,) = _sc_delta_kernel()(uidx, iidx, at)
    gu, gi = _sc_rows_kernel()(uidx, iidx, utab, itab)

    logit = _mlp(gu, gi, w_rows,
                 uidx.reshape(B, 1), iidx.reshape(B, 1),
                 W1, b1.reshape(1, 32), g1.reshape(1, 32),
                 be1.reshape(1, 32), Wl)
    return logit.reshape(B) + bl


# speedup vs baseline: 1.1996x; 1.1119x over previous
import functools

import jax
import jax.numpy as jnp
from jax import lax
from jax.experimental import pallas as pl
from jax.experimental.pallas import tpu as pltpu
from jax.experimental.pallas import tpu_sc as plsc

B = 4096
N_USERS = 10000
N_ITEMS = 5000
D = 16
TW = 128

SC_CORES = 2
SC_SUBCORES = 16
NW = SC_CORES * SC_SUBCORES
BPW = B // NW

ROW_BLK = 512
N_ROW_BLKS = (N_ITEMS + 1 + ROW_BLK - 1) // ROW_BLK
ROWS_PAD = N_ROW_BLKS * ROW_BLK
W_COLS = N_USERS + 1


def _sweep_body(a_ref, wiT_ref, wuT_ref, iproj_ref, uproj_ref):
    step = pl.program_id(0)
    a = a_ref[...]
    row_ids = step * ROW_BLK + lax.broadcasted_iota(jnp.int32, (ROW_BLK, 1), 0)
    a = jnp.where(row_ids < N_ITEMS, a, 0.0)
    iproj_ref[...] = jnp.dot(a, wiT_ref[...], preferred_element_type=jnp.float32)
    contrib = lax.dot_general(
        a, wuT_ref[...], (((0,), (0,)), ((), ())),
        preferred_element_type=jnp.float32)

    @pl.when(step == 0)
    def _():
        uproj_ref[...] = contrib

    @pl.when(step != 0)
    def _():
        uproj_ref[...] += contrib


def _projections(at, wiT_pad, wuT_pad):
    return pl.pallas_call(
        _sweep_body,
        grid=(N_ROW_BLKS,),
        in_specs=[
            pl.BlockSpec((ROW_BLK, W_COLS), lambda i: (i, 0)),
            pl.BlockSpec((W_COLS, D), lambda i: (0, 0)),
            pl.BlockSpec((ROW_BLK, D), lambda i: (i, 0)),
        ],
        out_specs=[
            pl.BlockSpec((ROW_BLK, D), lambda i: (i, 0)),
            pl.BlockSpec((W_COLS, D), lambda i: (0, 0)),
        ],
        out_shape=[
            jax.ShapeDtypeStruct((ROWS_PAD, D), jnp.float32),
            jax.ShapeDtypeStruct((W_COLS, D), jnp.float32),
        ],
        compiler_params=pltpu.CompilerParams(
            dimension_semantics=("arbitrary",)),
    )(at, wiT_pad, wuT_pad)


def _sc_delta_body(uidx_hbm, iidx_hbm, at_hbm, out_w,
                   uidx_v, iidx_v, w_v, wsem):
    wid = lax.axis_index("s") * SC_CORES + lax.axis_index("c")
    base = wid * BPW
    pltpu.sync_copy(uidx_hbm.at[pl.ds(base, BPW)], uidx_v)
    pltpu.sync_copy(iidx_hbm.at[pl.ds(base, BPW)], iidx_v)

    def chunk(j):
        off = pl.multiple_of(j * 16, 16)
        u16 = uidx_v[pl.ds(off, 16)]
        i16 = iidx_v[pl.ds(off, 16)]
        r0 = (i16 >> 3) << 3
        c0 = (u16 >> 7) << 7
        sub = i16 & 7
        waits = []
        for k in range(16):
            r_s = pl.multiple_of(r0[k], 8)
            c_s = pl.multiple_of(c0[k], 128)
            waits.append(pltpu.async_copy(
                at_hbm.at[pl.ds(r_s, 8), pl.ds(c_s, 128)],
                w_v.at[pl.ds(k * 8, 8)], wsem))
        for c in waits:
            c.wait()
        for k in range(16):
            pltpu.sync_copy(w_v.at[pl.ds(k * 8 + sub[k], 1)],
                            out_w.at[pl.ds(base + off + k, 1)])

    pl.loop(0, BPW // 16)(chunk)


@functools.cache
def _sc_delta_kernel():
    return functools.partial(
        pl.kernel,
        mesh=plsc.VectorSubcoreMesh(core_axis_name="c", subcore_axis_name="s"),
        out_type=[
            jax.ShapeDtypeStruct((B, TW), jnp.float32),
        ],
        scratch_types=[
            pltpu.VMEM((BPW,), jnp.int32),
            pltpu.VMEM((BPW,), jnp.int32),
            pltpu.VMEM((128, TW), jnp.float32),
            pltpu.SemaphoreType.DMA,
        ],
    )(_sc_delta_body)


def _sc_rows_body(uidx_hbm, iidx_hbm, utab_hbm, itab_hbm, order_hbm,
                  out_u, out_i,
                  uidx_v, iidx_v, r_u, r_i, sem):
    del order_hbm
    wid = lax.axis_index("s") * SC_CORES + lax.axis_index("c")
    base = wid * BPW
    pltpu.sync_copy(uidx_hbm.at[pl.ds(base, BPW)], uidx_v)
    pltpu.sync_copy(iidx_hbm.at[pl.ds(base, BPW)], iidx_v)

    cu = pltpu.async_copy(utab_hbm.at[uidx_v], r_u, sem)
    ci = pltpu.async_copy(itab_hbm.at[iidx_v], r_i, sem)
    cu.wait()
    ci.wait()
    pltpu.sync_copy(r_u, out_u.at[pl.ds(base, BPW)])
    pltpu.sync_copy(r_i, out_i.at[pl.ds(base, BPW)])


@functools.cache
def _sc_rows_kernel():
    return functools.partial(
        pl.kernel,
        mesh=plsc.VectorSubcoreMesh(core_axis_name="c", subcore_axis_name="s"),
        out_type=[
            jax.ShapeDtypeStruct((B, TW), jnp.float32),
            jax.ShapeDtypeStruct((B, TW), jnp.float32),
        ],
        scratch_types=[
            pltpu.VMEM((BPW,), jnp.int32),
            pltpu.VMEM((BPW,), jnp.int32),
            pltpu.VMEM((BPW, TW), jnp.float32),
            pltpu.VMEM((BPW, TW), jnp.float32),
            pltpu.SemaphoreType.DMA,
        ],
    )(_sc_rows_body)


def _mlp_body(gu_ref, gi_ref, w3_ref, ui_ref, ii_ref,
              w1_ref, b1_ref, g1_ref, be1_ref, wl_ref, out_ref):
    ui = ui_ref[...]
    ln = lax.broadcasted_iota(jnp.int32, (B, TW), 1)
    oh = ln == (ui & 127)
    delta = jnp.sum(jnp.where(oh, w3_ref[...], 0.0), axis=-1, keepdims=True)
    gu = gu_ref[...]
    gi = gi_ref[...]
    proj_user = gu[:, D:2 * D] - delta * gi[:, 2 * D:3 * D]
    proj_item = gi[:, D:2 * D] - delta * gu[:, 2 * D:3 * D]
    x = jnp.concatenate([gu[:, :D], proj_user, gi[:, :D], proj_item], axis=-1)
    h = lax.dot_general(x, w1_ref[...], (((1,), (1,)), ((), ())),
                        preferred_element_type=jnp.float32) + b1_ref[...]
    mu = jnp.mean(h, axis=-1, keepdims=True)
    var = jnp.mean(jnp.square(h - mu), axis=-1, keepdims=True)
    h = (h - mu) * lax.rsqrt(var + 1e-5) * g1_ref[...] + be1_ref[...]
    h = jnp.maximum(h, 0.0)
    out_ref[...] = lax.dot_general(h, wl_ref[...], (((1,), (1,)), ((), ())),
                                   preferred_element_type=jnp.float32)


def _mlp(gu, gi, w3, ui, ii, W1, b1, g1, be1, Wl):
    return pl.pallas_call(
        _mlp_body,
        out_shape=jax.ShapeDtypeStruct((B, 1), jnp.float32),
    )(gu, gi, w3, ui, ii, W1, b1, g1, be1, Wl)


def kernel(user_idx, item_idx, interactions, user_emb, item_emb, Wu, Wi,
           W1, b1, g1, be1, Wl, bl):
    at = interactions.T

    wiT_pad = jnp.zeros((W_COLS, D), jnp.float32).at[:N_USERS].set(Wi.T)
    wuT_pad = jnp.zeros((ROWS_PAD, D), jnp.float32).at[:N_ITEMS].set(Wu.T)

    iproj, uproj = _projections(at, wiT_pad, wuT_pad)

    zu = jnp.zeros((N_USERS, TW - 3 * D), jnp.float32)
    utab = jnp.concatenate(
        [user_emb[:N_USERS], uproj[:N_USERS], wiT_pad[:N_USERS], zu], axis=1)
    zi = jnp.zeros((N_ITEMS, TW - 3 * D), jnp.float32)
    itab = jnp.concatenate(
        [item_emb[:N_ITEMS], iproj[:N_ITEMS], wuT_pad[:N_ITEMS], zi], axis=1)

    uidx = user_idx.astype(jnp.int32)
    iidx = item_idx.astype(jnp.int32)
    (w_rows,) = _sc_delta_kernel()(uidx, iidx, at)
    gu, gi = _sc_rows_kernel()(uidx, iidx, utab, itab, w_rows)

    logit = _mlp(gu, gi, w_rows,
                 uidx.reshape(B, 1), iidx.reshape(B, 1),
                 W1, b1.reshape(1, 32), g1.reshape(1, 32),
                 be1.reshape(1, 32), Wl)
    return logit.reshape(B) + bl


# speedup vs baseline: 1.2082x; 1.0072x over previous
import functools

import jax
import jax.numpy as jnp
from jax import lax
from jax.experimental import pallas as pl
from jax.experimental.pallas import tpu as pltpu
from jax.experimental.pallas import tpu_sc as plsc

B = 4096
N_USERS = 10000
N_ITEMS = 5000
D = 16
TW = 128

SC_CORES = 2
SC_SUBCORES = 16
NW = SC_CORES * SC_SUBCORES
BPW = B // NW

ROW_BLK = 256
N_ROW_BLKS = (N_ITEMS + 1 + ROW_BLK - 1) // ROW_BLK
ROWS_PAD = N_ROW_BLKS * ROW_BLK
W_COLS = N_USERS + 1


def _sweep_body(a_ref, wiT_ref, wuT_ref, iproj_ref, uproj_ref, acc_ref):
    step = pl.program_id(0)
    a = a_ref[...]
    row_ids = step * ROW_BLK + lax.broadcasted_iota(jnp.int32, (ROW_BLK, 1), 0)
    a = jnp.where(row_ids < N_ITEMS, a, 0.0)
    ip = jnp.dot(a, wiT_ref[...], preferred_element_type=jnp.float32)
    iproj_ref[...] = jnp.concatenate(
        [ip, jnp.zeros((ROW_BLK, TW - D), jnp.float32)], axis=1)
    contrib = lax.dot_general(
        a, wuT_ref[...], (((0,), (0,)), ((), ())),
        preferred_element_type=jnp.float32)

    @pl.when(step == 0)
    def _():
        acc_ref[...] = contrib

    @pl.when(step != 0)
    def _():
        acc_ref[...] += contrib

    @pl.when(step == N_ROW_BLKS - 1)
    def _():
        uproj_ref[...] = jnp.concatenate(
            [acc_ref[...], jnp.zeros((W_COLS, TW - D), jnp.float32)], axis=1)


def _projections(at, wiT_pad, wuT_pad):
    return pl.pallas_call(
        _sweep_body,
        grid=(N_ROW_BLKS,),
        in_specs=[
            pl.BlockSpec((ROW_BLK, W_COLS), lambda i: (i, 0)),
            pl.BlockSpec((W_COLS, D), lambda i: (0, 0)),
            pl.BlockSpec((ROW_BLK, D), lambda i: (i, 0)),
        ],
        out_specs=[
            pl.BlockSpec((ROW_BLK, TW), lambda i: (i, 0)),
            pl.BlockSpec((W_COLS, TW), lambda i: (0, 0)),
        ],
        out_shape=[
            jax.ShapeDtypeStruct((ROWS_PAD, TW), jnp.float32),
            jax.ShapeDtypeStruct((W_COLS, TW), jnp.float32),
        ],
        scratch_shapes=[pltpu.VMEM((W_COLS, D), jnp.float32)],
        compiler_params=pltpu.CompilerParams(
            dimension_semantics=("arbitrary",)),
    )(at, wiT_pad, wuT_pad)


def _sc_delta_body(uidx_hbm, iidx_hbm, at_hbm, out_w,
                   uidx_v, iidx_v, w_v, wsem):
    wid = lax.axis_index("s") * SC_CORES + lax.axis_index("c")
    base = wid * BPW
    pltpu.sync_copy(uidx_hbm.at[pl.ds(base, BPW)], uidx_v)
    pltpu.sync_copy(iidx_hbm.at[pl.ds(base, BPW)], iidx_v)

    def chunk(j):
        off = pl.multiple_of(j * 16, 16)
        u16 = uidx_v[pl.ds(off, 16)]
        i16 = iidx_v[pl.ds(off, 16)]
        r0 = (i16 >> 3) << 3
        c0 = (u16 >> 7) << 7
        sub = i16 & 7
        waits = []
        for k in range(16):
            r_s = pl.multiple_of(r0[k], 8)
            c_s = pl.multiple_of(c0[k], 128)
            waits.append(pltpu.async_copy(
                at_hbm.at[pl.ds(r_s, 8), pl.ds(c_s, 128)],
                w_v.at[pl.ds(k * 8, 8)], wsem))
        for c in waits:
            c.wait()
        for k in range(16):
            pltpu.sync_copy(w_v.at[pl.ds(k * 8 + sub[k], 1)],
                            out_w.at[pl.ds(base + off + k, 1)])

    pl.loop(0, BPW // 16)(chunk)


@functools.cache
def _sc_delta_kernel():
    return functools.partial(
        pl.kernel,
        mesh=plsc.VectorSubcoreMesh(core_axis_name="c", subcore_axis_name="s"),
        out_type=[
            jax.ShapeDtypeStruct((B, TW), jnp.float32),
        ],
        scratch_types=[
            pltpu.VMEM((BPW,), jnp.int32),
            pltpu.VMEM((BPW,), jnp.int32),
            pltpu.VMEM((128, TW), jnp.float32),
            pltpu.SemaphoreType.DMA,
        ],
    )(_sc_delta_body)


def _sc_static_body(uidx_hbm, iidx_hbm, ustat_hbm, istat_hbm,
                    out_us, out_is,
                    uidx_v, iidx_v, r_us, r_is, sem):
    wid = lax.axis_index("s") * SC_CORES + lax.axis_index("c")
    base = wid * BPW
    pltpu.sync_copy(uidx_hbm.at[pl.ds(base, BPW)], uidx_v)
    pltpu.sync_copy(iidx_hbm.at[pl.ds(base, BPW)], iidx_v)

    cu = pltpu.async_copy(ustat_hbm.at[uidx_v], r_us, sem)
    ci = pltpu.async_copy(istat_hbm.at[iidx_v], r_is, sem)
    cu.wait()
    ci.wait()
    pltpu.sync_copy(r_us, out_us.at[pl.ds(base, BPW)])
    pltpu.sync_copy(r_is, out_is.at[pl.ds(base, BPW)])


@functools.cache
def _sc_static_kernel():
    return functools.partial(
        pl.kernel,
        mesh=plsc.VectorSubcoreMesh(core_axis_name="c", subcore_axis_name="s"),
        out_type=[
            jax.ShapeDtypeStruct((B, TW), jnp.float32),
            jax.ShapeDtypeStruct((B, TW), jnp.float32),
        ],
        scratch_types=[
            pltpu.VMEM((BPW,), jnp.int32),
            pltpu.VMEM((BPW,), jnp.int32),
            pltpu.VMEM((BPW, TW), jnp.float32),
            pltpu.VMEM((BPW, TW), jnp.float32),
            pltpu.SemaphoreType.DMA,
        ],
    )(_sc_static_body)


def _sc_proj_body(uidx_hbm, iidx_hbm, uproj_hbm, iproj_hbm, order_hbm,
                  out_up, out_ip,
                  uidx_v, iidx_v, r_up, r_ip, sem):
    del order_hbm
    wid = lax.axis_index("s") * SC_CORES + lax.axis_index("c")
    base = wid * BPW
    pltpu.sync_copy(uidx_hbm.at[pl.ds(base, BPW)], uidx_v)
    pltpu.sync_copy(iidx_hbm.at[pl.ds(base, BPW)], iidx_v)

    cu = pltpu.async_copy(uproj_hbm.at[uidx_v], r_up, sem)
    ci = pltpu.async_copy(iproj_hbm.at[iidx_v], r_ip, sem)
    cu.wait()
    ci.wait()
    pltpu.sync_copy(r_up, out_up.at[pl.ds(base, BPW)])
    pltpu.sync_copy(r_ip, out_ip.at[pl.ds(base, BPW)])


@functools.cache
def _sc_proj_kernel():
    return functools.partial(
        pl.kernel,
        mesh=plsc.VectorSubcoreMesh(core_axis_name="c", subcore_axis_name="s"),
        out_type=[
            jax.ShapeDtypeStruct((B, TW), jnp.float32),
            jax.ShapeDtypeStruct((B, TW), jnp.float32),
        ],
        scratch_types=[
            pltpu.VMEM((BPW,), jnp.int32),
            pltpu.VMEM((BPW,), jnp.int32),
            pltpu.VMEM((BPW, TW), jnp.float32),
            pltpu.VMEM((BPW, TW), jnp.float32),
            pltpu.SemaphoreType.DMA,
        ],
    )(_sc_proj_body)


def _mlp_body(gus_ref, gis_ref, gup_ref, gip_ref, w3_ref, ui_ref, ii_ref,
              w1_ref, b1_ref, g1_ref, be1_ref, wl_ref, out_ref):
    ui = ui_ref[...]
    ln = lax.broadcasted_iota(jnp.int32, (B, TW), 1)
    oh = ln == (ui & 127)
    delta = jnp.sum(jnp.where(oh, w3_ref[...], 0.0), axis=-1, keepdims=True)
    gus = gus_ref[...]
    gis = gis_ref[...]
    proj_user = gup_ref[:, :D] - delta * gis[:, D:2 * D]
    proj_item = gip_ref[:, :D] - delta * gus[:, D:2 * D]
    x = jnp.concatenate([gus[:, :D], proj_user, gis[:, :D], proj_item],
                        axis=-1)
    h = lax.dot_general(x, w1_ref[...], (((1,), (1,)), ((), ())),
                        preferred_element_type=jnp.float32) + b1_ref[...]
    mu = jnp.mean(h, axis=-1, keepdims=True)
    var = jnp.mean(jnp.square(h - mu), axis=-1, keepdims=True)
    h = (h - mu) * lax.rsqrt(var + 1e-5) * g1_ref[...] + be1_ref[...]
    h = jnp.maximum(h, 0.0)
    out_ref[...] = lax.dot_general(h, wl_ref[...], (((1,), (1,)), ((), ())),
                                   preferred_element_type=jnp.float32)


def _mlp(gus, gis, gup, gip, w3, ui, ii, W1, b1, g1, be1, Wl):
    return pl.pallas_call(
        _mlp_body,
        out_shape=jax.ShapeDtypeStruct((B, 1), jnp.float32),
    )(gus, gis, gup, gip, w3, ui, ii, W1, b1, g1, be1, Wl)


def kernel(user_idx, item_idx, interactions, user_emb, item_emb, Wu, Wi,
           W1, b1, g1, be1, Wl, bl):
    at = interactions.T

    wiT_pad = jnp.zeros((W_COLS, D), jnp.float32).at[:N_USERS].set(Wi.T)
    wuT_pad = jnp.zeros((ROWS_PAD, D), jnp.float32).at[:N_ITEMS].set(Wu.T)

    zu = jnp.zeros((N_USERS, TW - 2 * D), jnp.float32)
    ustat = jnp.concatenate(
        [user_emb[:N_USERS], wiT_pad[:N_USERS], zu], axis=1)
    zi = jnp.zeros((N_ITEMS, TW - 2 * D), jnp.float32)
    istat = jnp.concatenate(
        [item_emb[:N_ITEMS], wuT_pad[:N_ITEMS], zi], axis=1)

    uidx = user_idx.astype(jnp.int32)
    iidx = item_idx.astype(jnp.int32)
    (w_rows,) = _sc_delta_kernel()(uidx, iidx, at)
    gus, gis = _sc_static_kernel()(uidx, iidx, ustat, istat)

    iproj_tab, uproj_tab = _projections(at, wiT_pad, wuT_pad)
    gup, gip = _sc_proj_kernel()(uidx, iidx, uproj_tab, iproj_tab, w_rows)

    logit = _mlp(gus, gis, gup, gip, w_rows,
                 uidx.reshape(B, 1), iidx.reshape(B, 1),
                 W1, b1.reshape(1, 32), g1.reshape(1, 32),
                 be1.reshape(1, 32), Wl)
    return logit.reshape(B) + bl
